# baseline (speedup 1.0000x reference)
import functools

import numpy as np
import jax
import jax.numpy as jnp
from jax import lax
from jax.experimental import pallas as pl
from jax.experimental.pallas import tpu as pltpu, tpu_sc as plsc

_SIZE, _PATCH = 384, 16
_NUM = _SIZE // _PATCH
_P = _NUM * _NUM
_C = 32
_D = 768
_K = 8


def _static_indices():
    grid = np.arange(_P, dtype=np.int32).reshape(_NUM, _NUM)
    idx, inv = [], []
    for k in range(4):
        rot = np.rot90(grid, k=k, axes=(0, 1))
        for g in (rot, np.flip(rot, axis=1)):
            flat = g.flatten()
            idx.append(flat)
            inv.append(np.argsort(flat).astype(np.int32))
    return np.stack(idx), np.stack(inv)


_IDX, _ARGSORT = _static_indices()

_B = _K * _C * _P
_NC, _NS = 2, 16
_NW = _NC * _NS
_Q = 48
_NQ = _P // _Q
_NBUF = 3
_NTRIPLE = _NQ // _NBUF

_SIDX = (np.arange(_NW, dtype=np.int32)[:, None, None, None] * _P
         + np.arange(_K, dtype=np.int32)[None, None, :, None] * (_C * _P)
         + _ARGSORT.reshape(1, _K, _NQ, _Q).transpose(0, 2, 1, 3)
         ).reshape(_NW, _NQ * _K, _Q).astype(np.int32)


def _sc_augment(table, sidx):
    mesh = plsc.VectorSubcoreMesh(core_axis_name="c", subcore_axis_name="s")

    @functools.partial(
        pl.kernel,
        mesh=mesh,
        out_type=jax.ShapeDtypeStruct((_B, _D), jnp.float32),
        scratch_types=[
            pltpu.VMEM((_NQ * _K, _Q), jnp.int32),
            pltpu.VMEM((_Q, _D), jnp.float32),
            pltpu.VMEM((_Q, _D), jnp.float32),
            pltpu.VMEM((_Q, _D), jnp.float32),
            pltpu.SemaphoreType.DMA,
        ],
    )
    def aug_kernel(table_hbm, sidx_hbm, out_hbm, sidx_v,
                   buf0, buf1, buf2, ssem):
        wid = lax.axis_index("s") * _NC + lax.axis_index("c")
        pltpu.sync_copy(sidx_hbm.at[wid], sidx_v)
        in_base = wid * _P
        bufs = (buf0, buf1, buf2)

        def drain_one():
            pltpu.make_async_copy(table_hbm.at[pl.ds(0, _Q)], buf0, ssem).wait()

        def chunk_step(q, b):
            @pl.when(q >= _NBUF)
            def _():
                for _ in range(_K):
                    drain_one()
            pltpu.sync_copy(table_hbm.at[pl.ds(in_base + q * _Q, _Q)], bufs[b])
            pltpu.make_async_copy(
                bufs[b], out_hbm.at[pl.ds(in_base + q * _Q, _Q)], ssem
            ).start()
            for kk in range(1, _K):
                pltpu.make_async_copy(
                    bufs[b], out_hbm.at[sidx_v.at[q * _K + kk]], ssem
                ).start()

        def triple(j, carry):
            for u in range(_NBUF):
                chunk_step(_NBUF * j + u, u)
            return carry

        lax.fori_loop(0, _NTRIPLE, triple, 0)
        for _ in range(_NBUF * _K):
            drain_one()

    return aug_kernel(table, sidx)


def kernel(patch):
    table = patch.reshape(_C * _P, _D)
    out = _sc_augment(table, jnp.asarray(_SIDX))
    aug_tensor = out.reshape(_K, _C, _P, _D)
    argsort_tensor = jnp.asarray(_ARGSORT)
    perm = jnp.arange(_K, dtype=jnp.int32)
    return aug_tensor, argsort_tensor, perm


# speedup vs baseline: 1.0214x; 1.0214x over previous
import functools

import numpy as np
import jax
import jax.numpy as jnp
from jax import lax
from jax.experimental import pallas as pl
from jax.experimental.pallas import tpu as pltpu, tpu_sc as plsc

_SIZE, _PATCH = 384, 16
_NUM = _SIZE // _PATCH
_P = _NUM * _NUM
_C = 32
_D = 768
_K = 8


def _static_indices():
    grid = np.arange(_P, dtype=np.int32).reshape(_NUM, _NUM)
    idx, inv = [], []
    for k in range(4):
        rot = np.rot90(grid, k=k, axes=(0, 1))
        for g in (rot, np.flip(rot, axis=1)):
            flat = g.flatten()
            idx.append(flat)
            inv.append(np.argsort(flat).astype(np.int32))
    return np.stack(idx), np.stack(inv)


_IDX, _ARGSORT = _static_indices()

_B = _K * _C * _P
_NC, _NS = 2, 16
_NW = _NC * _NS
_Q = 72
_NQ = _P // _Q
_NBUF = 2
_NGROUP = _NQ // _NBUF

_SIDX = (np.arange(_NW, dtype=np.int32)[:, None, None, None] * _P
         + np.arange(_K, dtype=np.int32)[None, None, :, None] * (_C * _P)
         + _ARGSORT.reshape(1, _K, _NQ, _Q).transpose(0, 2, 1, 3)
         ).reshape(_NW, _NQ * _K, _Q).astype(np.int32)


def _sc_augment(table, sidx):
    mesh = plsc.VectorSubcoreMesh(core_axis_name="c", subcore_axis_name="s")

    @functools.partial(
        pl.kernel,
        mesh=mesh,
        out_type=jax.ShapeDtypeStruct((_B, _D), jnp.float32),
        scratch_types=[
            pltpu.VMEM((_NQ * _K, _Q), jnp.int32),
            pltpu.VMEM((_Q, _D), jnp.float32),
            pltpu.VMEM((_Q, _D), jnp.float32),
            pltpu.SemaphoreType.DMA,
        ],
    )
    def aug_kernel(table_hbm, sidx_hbm, out_hbm, sidx_v, buf0, buf1, ssem):
        wid = lax.axis_index("s") * _NC + lax.axis_index("c")
        pltpu.sync_copy(sidx_hbm.at[wid], sidx_v)
        in_base = wid * _P
        bufs = (buf0, buf1)

        def drain_one():
            pltpu.make_async_copy(table_hbm.at[pl.ds(0, _Q)], buf0, ssem).wait()

        def chunk_step(q, b):
            @pl.when(q >= _NBUF)
            def _():
                for _ in range(_K):
                    drain_one()
            pltpu.sync_copy(table_hbm.at[pl.ds(in_base + q * _Q, _Q)], bufs[b])
            pltpu.make_async_copy(
                bufs[b], out_hbm.at[pl.ds(in_base + q * _Q, _Q)], ssem
            ).start()
            for kk in range(1, _K):
                pltpu.make_async_copy(
                    bufs[b], out_hbm.at[sidx_v.at[q * _K + kk]], ssem
                ).start()

        def group(j, carry):
            for u in range(_NBUF):
                chunk_step(_NBUF * j + u, u)
            return carry

        lax.fori_loop(0, _NGROUP, group, 0)
        for _ in range(_NBUF * _K):
            drain_one()

    return aug_kernel(table, sidx)


def kernel(patch):
    table = patch.reshape(_C * _P, _D)
    out = _sc_augment(table, jnp.asarray(_SIDX))
    aug_tensor = out.reshape(_K, _C, _P, _D)
    argsort_tensor = jnp.asarray(_ARGSORT)
    perm = jnp.arange(_K, dtype=jnp.int32)
    return aug_tensor, argsort_tensor, perm


# speedup vs baseline: 1.0729x; 1.0504x over previous
import functools

import numpy as np
import jax
import jax.numpy as jnp
from jax import lax
from jax.experimental import pallas as pl
from jax.experimental.pallas import tpu as pltpu, tpu_sc as plsc

_SIZE, _PATCH = 384, 16
_NUM = _SIZE // _PATCH
_P = _NUM * _NUM
_C = 32
_D = 768
_K = 8


def _static_indices():
    grid = np.arange(_P, dtype=np.int32).reshape(_NUM, _NUM)
    idx, inv = [], []
    for k in range(4):
        rot = np.rot90(grid, k=k, axes=(0, 1))
        for g in (rot, np.flip(rot, axis=1)):
            flat = g.flatten()
            idx.append(flat)
            inv.append(np.argsort(flat).astype(np.int32))
    return np.stack(idx), np.stack(inv)


_IDX, _ARGSORT = _static_indices()

_B = _K * _C * _P
_NC, _NS = 2, 16
_NW = _NC * _NS
_Q = 96
_NQ = _P // _Q
_NBUF = 1
_NGROUP = _NQ // _NBUF

_SIDX = (np.arange(_NW, dtype=np.int32)[:, None, None, None] * _P
         + np.arange(_K, dtype=np.int32)[None, None, :, None] * (_C * _P)
         + _ARGSORT.reshape(1, _K, _NQ, _Q).transpose(0, 2, 1, 3)
         ).reshape(_NW, _NQ * _K, _Q).astype(np.int32)


def _sc_augment(table, sidx):
    mesh = plsc.VectorSubcoreMesh(core_axis_name="c", subcore_axis_name="s")

    @functools.partial(
        pl.kernel,
        mesh=mesh,
        out_type=jax.ShapeDtypeStruct((_B, _D), jnp.float32),
        scratch_types=[
            pltpu.VMEM((_NQ * _K, _Q), jnp.int32),
            pltpu.VMEM((_Q, _D), jnp.float32),
            pltpu.SemaphoreType.DMA,
        ],
    )
    def aug_kernel(table_hbm, sidx_hbm, out_hbm, sidx_v, buf0, ssem):
        wid = lax.axis_index("s") * _NC + lax.axis_index("c")
        pltpu.sync_copy(sidx_hbm.at[wid], sidx_v)
        in_base = wid * _P
        bufs = (buf0,)

        def drain_one():
            pltpu.make_async_copy(table_hbm.at[pl.ds(0, _Q)], buf0, ssem).wait()

        def chunk_step(q, b):
            @pl.when(q >= _NBUF)
            def _():
                for _ in range(_K):
                    drain_one()
            pltpu.sync_copy(table_hbm.at[pl.ds(in_base + q * _Q, _Q)], bufs[b])
            pltpu.make_async_copy(
                bufs[b], out_hbm.at[pl.ds(in_base + q * _Q, _Q)], ssem
            ).start()
            for kk in range(1, _K):
                pltpu.make_async_copy(
                    bufs[b], out_hbm.at[sidx_v.at[q * _K + kk]], ssem
                ).start()

        def group(j, carry):
            for u in range(_NBUF):
                chunk_step(_NBUF * j + u, u)
            return carry

        lax.fori_loop(0, _NGROUP, group, 0)
        for _ in range(_NBUF * _K):
            drain_one()

    return aug_kernel(table, sidx)


def kernel(patch):
    table = patch.reshape(_C * _P, _D)
    out = _sc_augment(table, jnp.asarray(_SIDX))
    aug_tensor = out.reshape(_K, _C, _P, _D)
    argsort_tensor = jnp.asarray(_ARGSORT)
    perm = jnp.arange(_K, dtype=jnp.int32)
    return aug_tensor, argsort_tensor, perm
